# baseline (speedup 1.0000x reference)
import functools

import jax
import jax.numpy as jnp
from jax import lax
from jax.experimental import pallas as pl
from jax.experimental.pallas import tpu as pltpu
from jax.experimental.pallas import tpu_sc as plsc


def _score_body(x_ref, kp_ref, q_ref, o_ref, w_ref):
    @pl.when(pl.program_id(0) == 0)
    def _():
        w_ref[...] = jnp.dot(q_ref[...], kp_ref[...],
                             preferred_element_type=jnp.float32)
    s = lax.dot_general(w_ref[...], x_ref[...],
                        (((1,), (1,)), ((), ())),
                        preferred_element_type=jnp.float32)
    o_ref[...] = jnp.max(s, axis=0)[None, None, :]


_I32_MIN = -(2 ** 31)


def _pick_unit(totals, need_s, lanes):
    rev_t = lax.rev(totals, (0,))
    csum = plsc.cumsum(rev_t)
    ge = csum >= jnp.broadcast_to(need_s, (16,))
    j0 = plsc.all_reduce_ffs(ge)
    unit_v = 15 - j0
    above = jnp.sum(jnp.where(lanes == j0 - 1, csum, 0))
    return jnp.max(unit_v), need_s - above


def _make_topk(rows, n, k, out_pad):
    nch = 8
    ch = n // nch
    mesh = plsc.VectorSubcoreMesh(core_axis_name="c", subcore_axis_name="s")

    @functools.partial(
        pl.kernel,
        mesh=mesh,
        compiler_params=pltpu.CompilerParams(needs_layout_passes=False),
        out_type=jax.ShapeDtypeStruct((rows, out_pad), jnp.int32),
        scratch_types=[
            pltpu.VMEM((ch,), jnp.float32),
            pltpu.VMEM((ch,), jnp.int32),
            pltpu.VMEM((ch,), jnp.int32),
            pltpu.VMEM((ch,), jnp.int32),
            pltpu.VMEM((256 * 16,), jnp.int32),
            pltpu.VMEM((256,), jnp.int32),
            pltpu.VMEM((nch * 256,), jnp.int32),
            pltpu.VMEM((nch * 16,), jnp.int32),
            pltpu.VMEM((16,), jnp.int32),
            pltpu.VMEM((n,), jnp.int32),
            pltpu.VMEM((n,), jnp.int32),
            pltpu.VMEM((n + 32,), jnp.int32),
            pltpu.VMEM((n + 32,), jnp.int32),
            pltpu.VMEM((n + 32,), jnp.int32),
            pltpu.VMEM((out_pad,), jnp.int32),
            pltpu.VMEM_SHARED((2 * nch * 256,), jnp.int32),
            pltpu.VMEM_SHARED((2 * nch * 16,), jnp.int32),
            pltpu.VMEM_SHARED((2 * nch * ch,), jnp.int32),
            pltpu.VMEM_SHARED((2 * nch * ch,), jnp.int32),
        ],
    )
    def topk(scores_hbm, out_hbm, vals_v, keys_v, lkey_v, lidx_v, hist_v,
             tot_v, comb_v, cnts_v, pub_v, ckey_v, cidx_v, ckey2_v, cidx2_v,
             akey_v, out_v, tot_sh, cnt_sh, candk_sh, candi_sh):
        cid = lax.axis_index("c")
        sid = lax.axis_index("s")
        lr = sid // nch
        chunk = sid % nch
        b = cid * 2 + lr

        lanes = lax.iota(jnp.int32, 16)
        ones = jnp.ones((16,), jnp.int32)
        zeros = jnp.zeros((16,), jnp.int32)
        topbit = jnp.full((16,), _I32_MIN, jnp.int32)
        sh24 = jnp.full((16,), 24, jnp.int32)

        scope1 = jax.named_scope("ph1_hist")
        scope1.__enter__()
        pltpu.sync_copy(scores_hbm.at[b, pl.ds(chunk * ch, ch)], vals_v)

        def zbody(i, carry):
            for u in range(4):
                hist_v[pl.ds(i * 64 + u * 16, 16)] = zeros
            return carry
        lax.fori_loop(0, 64, zbody, 0)

        def pa(i, carry):
            for u in range(2):
                off = i * 32 + u * 16
                v = vals_v[pl.ds(off, 16)] + 0.0
                bi = lax.bitcast_convert_type(v, jnp.int32)
                skey = jnp.where(bi < 0, bi ^ jnp.int32(0x7FFFFFFF), bi)
                keys_v[pl.ds(off, 16)] = skey
                dig = lax.shift_right_logical(skey ^ topbit, sh24)
                plsc.addupdate_scatter(hist_v, [dig * 16 + lanes], ones)
            return carry
        lax.fori_loop(0, ch // 32, pa, 0)

        for g in range(16):
            acc = zeros
            for l in range(16):
                acc = acc + plsc.load_gather(
                    hist_v, [(g * 16 + lanes) * 16 + l])
            tot_v[pl.ds(g * 16, 16)] = acc
        pltpu.sync_copy(tot_v, tot_sh.at[pl.ds((lr * nch + chunk) * 256, 256)])
        scope1.__exit__(None, None, None)

        plsc.subcore_barrier()

        scope2 = jax.named_scope("ph2_compact")
        scope2.__enter__()
        pltpu.sync_copy(tot_sh.at[pl.ds(lr * (nch * 256), nch * 256)], comb_v)
        need = jnp.int32(k)
        gsums = zeros
        for g in range(16):
            acc = comb_v[pl.ds(g * 16, 16)]
            for j in range(1, nch):
                acc = acc + comb_v[pl.ds(j * 256 + g * 16, 16)]
            tot_v[pl.ds(g * 16, 16)] = acc
            gsums = jnp.where(lanes == g,
                              jnp.broadcast_to(jnp.sum(acc), (16,)), gsums)
        gsel, need = _pick_unit(gsums, need, lanes)
        bin_tot = tot_v[pl.ds(gsel * 16, 16)]
        bsel, need = _pick_unit(bin_tot, need, lanes)
        bsel0 = gsel * 16 + bsel

        t0 = jnp.broadcast_to(
            lax.shift_left(bsel0, jnp.int32(24)) ^ jnp.int32(_I32_MIN),
            (16,))
        def pb(i, cnt_v):
            for u in range(2):
                off = i * 32 + u * 16
                skey = keys_v[pl.ds(off, 16)]
                m = skey >= t0
                pos = plsc.cumsum(m.astype(jnp.int32)) - 1 + cnt_v
                plsc.store_scatter(lidx_v, [pos], lanes + chunk * ch + off,
                                   mask=m)
                plsc.store_scatter(lkey_v, [pos], skey, mask=m)
                cnt_v = cnt_v + plsc.all_reduce_population_count(m)
            return cnt_v
        cnt_v = lax.fori_loop(0, ch // 32, pb, zeros)
        pub_v[pl.ds(0, 16)] = cnt_v
        pltpu.sync_copy(pub_v, cnt_sh.at[pl.ds((lr * nch + chunk) * 16, 16)])
        pltpu.sync_copy(lkey_v,
                        candk_sh.at[pl.ds((lr * nch + chunk) * ch, ch)])
        pltpu.sync_copy(lidx_v,
                        candi_sh.at[pl.ds((lr * nch + chunk) * ch, ch)])
        scope2.__exit__(None, None, None)

        plsc.subcore_barrier()

        @pl.when(chunk == 0)
        def _():
            scope3 = jax.named_scope("ph3a_fetch")
            scope3.__enter__()
            pltpu.sync_copy(cnt_sh.at[pl.ds(lr * (nch * 16), nch * 16)],
                            cnts_v)
            pltpu.sync_copy(candk_sh.at[pl.ds(lr * n, n)], ckey_v)
            pltpu.sync_copy(candi_sh.at[pl.ds(lr * n, n)], cidx_v)
            cvec = zeros
            for j in range(nch):
                cvec = jnp.where(lanes == j, cnts_v[pl.ds(j * 16, 16)], cvec)
            cum = plsc.cumsum(cvec)
            base_vec = cum - cvec
            total_s = jnp.max(cum)
            cnt_all = jnp.broadcast_to(total_s, (16,))
            cj = []
            for j in range(nch - 1):
                cj.append(jnp.broadcast_to(
                    jnp.sum(jnp.where(lanes == j, cum, 0)), (16,)))
            hist_v[pl.ds(0, 16)] = base_vec
            ncand = (total_s + 15) // 16
            lim = jnp.full((16,), n - 1, jnp.int32)

            scope3.__exit__(None, None, None)
            scope3b = jax.named_scope("ph3b_repack")
            scope3b.__enter__()

            mask_f = jnp.full((16,), 0xF, jnp.int32)
            sh20 = jnp.full((16,), 20, jnp.int32)
            for j in range(16):
                hist_v[pl.ds(16 + j * 16, 16)] = zeros
            bsel0_v = jnp.broadcast_to(bsel0, (16,))

            def rp(i, carry):
                for u in range(2):
                    pvec = lanes + (i * 32 + u * 16)
                    region = zeros
                    for j in range(nch - 1):
                        region = region + (pvec >= cj[j]).astype(jnp.int32)
                    gbase = plsc.load_gather(hist_v, [region])
                    src = jnp.minimum(region * ch + (pvec - gbase), lim)
                    kk = plsc.load_gather(ckey_v, [src])
                    ckey2_v[pl.ds(i * 32 + u * 16, 16)] = kk
                    cidx2_v[pl.ds(i * 32 + u * 16, 16)] = (
                        plsc.load_gather(cidx_v, [src]))
                    ukey = kk ^ topbit
                    m = jnp.logical_and(
                        pvec < cnt_all,
                        lax.shift_right_logical(ukey, sh24) == bsel0_v)
                    dig = lax.shift_right_logical(ukey, sh20) & mask_f
                    plsc.addupdate_scatter(hist_v, [16 + dig * 16 + lanes],
                                           ones, mask=m)
                return carry
            lax.fori_loop(0, (ncand + 1) // 2, rp, 0)
            cnt_v = cnt_all
            scope3b.__exit__(None, None, None)
            scope3c = jax.named_scope("ph3c_refine")
            scope3c.__enter__()

            def _scan16(nd):
                bt = zeros
                for l in range(16):
                    bt = bt + plsc.load_gather(hist_v,
                                               [16 + lanes * 16 + l])
                return _pick_unit(bt, nd, lanes)

            need_l = need
            bsel_q, need_l = _scan16(need_l)
            prefix = lax.shift_left(bsel0, jnp.int32(4)) | bsel_q

            acnt_v = zeros
            ncur = ncand
            for q in range(5):
                shift = 16 - 4 * q
                for j in range(16):
                    hist_v[pl.ds(16 + j * 16, 16)] = zeros
                prefix_v = jnp.broadcast_to(prefix, (16,))
                sh_m = jnp.full((16,), shift + 4, jnp.int32)
                sh_d = jnp.full((16,), shift, jnp.int32)
                pcnt_v = cnt_v if q == 0 else acnt_v
                sref = ckey2_v if q == 0 else akey_v

                def cq(i, ac, _s=sref, _pc=pcnt_v, _sm=sh_m, _sd=sh_d,
                       _pv=prefix_v):
                    kk = _s[pl.ds(i * 16, 16)]
                    ukey = kk ^ topbit
                    valid = (lanes + i * 16) < _pc
                    m = jnp.logical_and(
                        valid, lax.shift_right_logical(ukey, _sm) == _pv)
                    pos = plsc.cumsum(m.astype(jnp.int32)) - 1 + ac
                    plsc.store_scatter(akey_v, [pos], kk, mask=m)
                    ac = ac + plsc.all_reduce_population_count(m)
                    dig = lax.shift_right_logical(ukey, _sd) & mask_f
                    plsc.addupdate_scatter(hist_v,
                                           [16 + dig * 16 + lanes], ones,
                                           mask=m)
                    return ac
                acnt_v = lax.fori_loop(0, ncur, cq, zeros)
                ncur = (jnp.max(acnt_v) + 15) // 16
                bsel_q, need_l = _scan16(need_l)
                prefix = lax.shift_left(prefix, jnp.int32(4)) | bsel_q

            scope3c.__exit__(None, None, None)
            scope3d = jax.named_scope("ph3d_final")
            scope3d.__enter__()
            t_vec = jnp.broadcast_to(prefix ^ jnp.int32(_I32_MIN), (16,))

            out_v[pl.ds(out_pad - 16, 16)] = zeros

            def fl(i, carry):
                cursor_v, budget_v = carry
                for u in range(2):
                    off = i * 32 + u * 16
                    valid = (lanes + off) < cnt_v
                    kk = ckey2_v[pl.ds(off, 16)]
                    gt = jnp.logical_and(valid, kk > t_vec)
                    eq = jnp.logical_and(valid, kk == t_vec)
                    eqc = plsc.cumsum(eq.astype(jnp.int32))
                    take_eq = jnp.logical_and(eq, eqc <= budget_v)
                    take = jnp.logical_or(gt, take_eq)
                    pos = plsc.cumsum(take.astype(jnp.int32)) - 1 + cursor_v
                    idxs = cidx2_v[pl.ds(off, 16)]
                    plsc.store_scatter(out_v, [pos], idxs, mask=take)
                    cursor_v = (cursor_v
                                + plsc.all_reduce_population_count(take))
                    budget_v = (budget_v
                                - plsc.all_reduce_population_count(take_eq))
                return (cursor_v, budget_v)
            lax.fori_loop(0, (ncand + 1) // 2, fl,
                          (zeros, jnp.broadcast_to(need_l, (16,))))

            pltpu.sync_copy(out_v, out_hbm.at[b])
            scope3d.__exit__(None, None, None)

    return topk


def kernel(x, k_proj_w, queries):
    b, n, d = x.shape
    k = max(1, int(n * 0.1))
    nq, r = queries.shape

    blk = 1024
    rows_total = b * n
    nb = rows_total // blk
    x2 = x.reshape(rows_total, d)
    ts3 = pl.pallas_call(
        _score_body,
        grid=(nb,),
        in_specs=[
            pl.BlockSpec((blk, d), lambda i: (i, 0)),
            pl.BlockSpec((r, d), lambda i: (0, 0)),
            pl.BlockSpec((nq, r), lambda i: (0, 0)),
        ],
        out_specs=pl.BlockSpec((1, 1, blk), lambda i: (i, 0, 0)),
        out_shape=jax.ShapeDtypeStruct((nb, 1, blk), jnp.float32),
        scratch_shapes=[pltpu.VMEM((nq, d), jnp.float32)],
    )(x2, k_proj_w, queries)
    ts = ts3.reshape(b, n)

    out_pad = ((k + 7) // 8) * 8
    out = _make_topk(b, n, k, out_pad)(ts)
    return out[:, :k]


# speedup vs baseline: 1.0031x; 1.0031x over previous
import functools

import jax
import jax.numpy as jnp
from jax import lax
from jax.experimental import pallas as pl
from jax.experimental.pallas import tpu as pltpu
from jax.experimental.pallas import tpu_sc as plsc


def _score_body(x_ref, kp_ref, q_ref, o_ref, w_ref):
    @pl.when(pl.program_id(0) == 0)
    def _():
        w_ref[...] = jnp.dot(q_ref[...], kp_ref[...],
                             preferred_element_type=jnp.float32)
    s = lax.dot_general(w_ref[...], x_ref[...],
                        (((1,), (1,)), ((), ())),
                        preferred_element_type=jnp.float32)
    o_ref[...] = jnp.max(s, axis=0)[None, None, :]


_I32_MIN = -(2 ** 31)


def _pick_unit(totals, need_s, lanes):
    rev_t = lax.rev(totals, (0,))
    csum = plsc.cumsum(rev_t)
    ge = csum >= jnp.broadcast_to(need_s, (16,))
    j0 = plsc.all_reduce_ffs(ge)
    unit_v = 15 - j0
    above = jnp.sum(jnp.where(lanes == j0 - 1, csum, 0))
    return jnp.max(unit_v), need_s - above


def _make_topk(rows, n, k, out_pad):
    nch = 8
    ch = n // nch
    mesh = plsc.VectorSubcoreMesh(core_axis_name="c", subcore_axis_name="s")

    @functools.partial(
        pl.kernel,
        mesh=mesh,
        compiler_params=pltpu.CompilerParams(needs_layout_passes=False),
        out_type=jax.ShapeDtypeStruct((rows, out_pad), jnp.int32),
        scratch_types=[
            pltpu.VMEM((ch,), jnp.float32),
            pltpu.VMEM((ch,), jnp.int32),
            pltpu.VMEM((ch,), jnp.int32),
            pltpu.VMEM((ch,), jnp.int32),
            pltpu.VMEM((256 * 16,), jnp.int32),
            pltpu.VMEM((256,), jnp.int32),
            pltpu.VMEM((nch * 256,), jnp.int32),
            pltpu.VMEM((nch * 16,), jnp.int32),
            pltpu.VMEM((16,), jnp.int32),
            pltpu.VMEM((n,), jnp.int32),
            pltpu.VMEM((n,), jnp.int32),
            pltpu.VMEM((n + 32,), jnp.int32),
            pltpu.VMEM((n + 32,), jnp.int32),
            pltpu.VMEM((n + 32,), jnp.int32),
            pltpu.VMEM((out_pad,), jnp.int32),
            pltpu.VMEM_SHARED((2 * nch * 256,), jnp.int32),
            pltpu.VMEM_SHARED((2 * nch * 16,), jnp.int32),
            pltpu.VMEM_SHARED((2 * nch * ch,), jnp.int32),
            pltpu.VMEM_SHARED((2 * nch * ch,), jnp.int32),
        ],
    )
    def topk(scores_hbm, out_hbm, vals_v, keys_v, lkey_v, lidx_v, hist_v,
             tot_v, comb_v, cnts_v, pub_v, ckey_v, cidx_v, ckey2_v, cidx2_v,
             akey_v, out_v, tot_sh, cnt_sh, candk_sh, candi_sh):
        cid = lax.axis_index("c")
        sid = lax.axis_index("s")
        lr = sid // nch
        chunk = sid % nch
        b = cid * 2 + lr

        lanes = lax.iota(jnp.int32, 16)
        ones = jnp.ones((16,), jnp.int32)
        zeros = jnp.zeros((16,), jnp.int32)
        topbit = jnp.full((16,), _I32_MIN, jnp.int32)
        sh24 = jnp.full((16,), 24, jnp.int32)

        pltpu.sync_copy(scores_hbm.at[b, pl.ds(chunk * ch, ch)], vals_v)

        def zbody(i, carry):
            for u in range(4):
                hist_v[pl.ds(i * 64 + u * 16, 16)] = zeros
            return carry
        lax.fori_loop(0, 64, zbody, 0)

        def pa(i, carry):
            for u in range(2):
                off = i * 32 + u * 16
                v = vals_v[pl.ds(off, 16)] + 0.0
                bi = lax.bitcast_convert_type(v, jnp.int32)
                skey = jnp.where(bi < 0, bi ^ jnp.int32(0x7FFFFFFF), bi)
                keys_v[pl.ds(off, 16)] = skey
                dig = lax.shift_right_logical(skey ^ topbit, sh24)
                plsc.addupdate_scatter(hist_v, [dig * 16 + lanes], ones)
            return carry
        lax.fori_loop(0, ch // 32, pa, 0)

        for g in range(16):
            acc = zeros
            for l in range(16):
                acc = acc + plsc.load_gather(
                    hist_v, [(g * 16 + lanes) * 16 + l])
            tot_v[pl.ds(g * 16, 16)] = acc
        pltpu.sync_copy(tot_v, tot_sh.at[pl.ds((lr * nch + chunk) * 256, 256)])

        plsc.subcore_barrier()

        pltpu.sync_copy(tot_sh.at[pl.ds(lr * (nch * 256), nch * 256)], comb_v)
        need = jnp.int32(k)
        gsums = zeros
        for g in range(16):
            acc = comb_v[pl.ds(g * 16, 16)]
            for j in range(1, nch):
                acc = acc + comb_v[pl.ds(j * 256 + g * 16, 16)]
            tot_v[pl.ds(g * 16, 16)] = acc
            gsums = jnp.where(lanes == g,
                              jnp.broadcast_to(jnp.sum(acc), (16,)), gsums)
        gsel, need = _pick_unit(gsums, need, lanes)
        bin_tot = tot_v[pl.ds(gsel * 16, 16)]
        bsel, need = _pick_unit(bin_tot, need, lanes)
        bsel0 = gsel * 16 + bsel

        t0 = jnp.broadcast_to(
            lax.shift_left(bsel0, jnp.int32(24)) ^ jnp.int32(_I32_MIN),
            (16,))
        def pb(i, cnt_v):
            for u in range(2):
                off = i * 32 + u * 16
                skey = keys_v[pl.ds(off, 16)]
                m = skey >= t0
                pos = plsc.cumsum(m.astype(jnp.int32)) - 1 + cnt_v
                plsc.store_scatter(lidx_v, [pos], lanes + chunk * ch + off,
                                   mask=m)
                plsc.store_scatter(lkey_v, [pos], skey, mask=m)
                cnt_v = cnt_v + plsc.all_reduce_population_count(m)
            return cnt_v
        cnt_v = lax.fori_loop(0, ch // 32, pb, zeros)
        pub_v[pl.ds(0, 16)] = cnt_v
        pltpu.sync_copy(pub_v, cnt_sh.at[pl.ds((lr * nch + chunk) * 16, 16)])
        pltpu.sync_copy(lkey_v,
                        candk_sh.at[pl.ds((lr * nch + chunk) * ch, ch)])
        pltpu.sync_copy(lidx_v,
                        candi_sh.at[pl.ds((lr * nch + chunk) * ch, ch)])

        plsc.subcore_barrier()

        @pl.when(chunk == 0)
        def _():
            pltpu.sync_copy(cnt_sh.at[pl.ds(lr * (nch * 16), nch * 16)],
                            cnts_v)
            pltpu.sync_copy(candk_sh.at[pl.ds(lr * n, n)], ckey_v)
            pltpu.sync_copy(candi_sh.at[pl.ds(lr * n, n)], cidx_v)
            cvec = zeros
            for j in range(nch):
                cvec = jnp.where(lanes == j, cnts_v[pl.ds(j * 16, 16)], cvec)
            cum = plsc.cumsum(cvec)
            base_vec = cum - cvec
            total_s = jnp.max(cum)
            cnt_all = jnp.broadcast_to(total_s, (16,))
            cj = []
            for j in range(nch - 1):
                cj.append(jnp.broadcast_to(
                    jnp.sum(jnp.where(lanes == j, cum, 0)), (16,)))
            hist_v[pl.ds(0, 16)] = base_vec
            ncand = (total_s + 15) // 16
            lim = jnp.full((16,), n - 1, jnp.int32)


            mask_f = jnp.full((16,), 0xF, jnp.int32)
            sh20 = jnp.full((16,), 20, jnp.int32)
            for j in range(16):
                hist_v[pl.ds(16 + j * 16, 16)] = zeros
            bsel0_v = jnp.broadcast_to(bsel0, (16,))

            def rp(i, carry):
                for u in range(2):
                    pvec = lanes + (i * 32 + u * 16)
                    region = zeros
                    for j in range(nch - 1):
                        region = region + (pvec >= cj[j]).astype(jnp.int32)
                    gbase = plsc.load_gather(hist_v, [region])
                    src = jnp.minimum(region * ch + (pvec - gbase), lim)
                    kk = plsc.load_gather(ckey_v, [src])
                    ckey2_v[pl.ds(i * 32 + u * 16, 16)] = kk
                    cidx2_v[pl.ds(i * 32 + u * 16, 16)] = (
                        plsc.load_gather(cidx_v, [src]))
                    ukey = kk ^ topbit
                    m = jnp.logical_and(
                        pvec < cnt_all,
                        lax.shift_right_logical(ukey, sh24) == bsel0_v)
                    dig = lax.shift_right_logical(ukey, sh20) & mask_f
                    plsc.addupdate_scatter(hist_v, [16 + dig * 16 + lanes],
                                           ones, mask=m)
                return carry
            lax.fori_loop(0, (ncand + 1) // 2, rp, 0)
            cnt_v = cnt_all

            def _scan16(nd):
                bt = zeros
                for l in range(16):
                    bt = bt + plsc.load_gather(hist_v,
                                               [16 + lanes * 16 + l])
                return _pick_unit(bt, nd, lanes)

            need_l = need
            bsel_q, need_l = _scan16(need_l)
            prefix = lax.shift_left(bsel0, jnp.int32(4)) | bsel_q

            acnt_v = zeros
            ncur = ncand
            for q in range(5):
                shift = 16 - 4 * q
                for j in range(16):
                    hist_v[pl.ds(16 + j * 16, 16)] = zeros
                prefix_v = jnp.broadcast_to(prefix, (16,))
                sh_m = jnp.full((16,), shift + 4, jnp.int32)
                sh_d = jnp.full((16,), shift, jnp.int32)
                pcnt_v = cnt_v if q == 0 else acnt_v
                sref = ckey2_v if q == 0 else akey_v

                def cq(i, ac, _s=sref, _pc=pcnt_v, _sm=sh_m, _sd=sh_d,
                       _pv=prefix_v):
                    kk = _s[pl.ds(i * 16, 16)]
                    ukey = kk ^ topbit
                    valid = (lanes + i * 16) < _pc
                    m = jnp.logical_and(
                        valid, lax.shift_right_logical(ukey, _sm) == _pv)
                    pos = plsc.cumsum(m.astype(jnp.int32)) - 1 + ac
                    plsc.store_scatter(akey_v, [pos], kk, mask=m)
                    ac = ac + plsc.all_reduce_population_count(m)
                    dig = lax.shift_right_logical(ukey, _sd) & mask_f
                    plsc.addupdate_scatter(hist_v,
                                           [16 + dig * 16 + lanes], ones,
                                           mask=m)
                    return ac
                acnt_v = lax.fori_loop(0, ncur, cq, zeros)
                ncur = (jnp.max(acnt_v) + 15) // 16
                bsel_q, need_l = _scan16(need_l)
                prefix = lax.shift_left(prefix, jnp.int32(4)) | bsel_q

            t_vec = jnp.broadcast_to(prefix ^ jnp.int32(_I32_MIN), (16,))

            out_v[pl.ds(out_pad - 16, 16)] = zeros

            def fl(i, carry):
                cursor_v, budget_v = carry
                for u in range(2):
                    off = i * 32 + u * 16
                    valid = (lanes + off) < cnt_v
                    kk = ckey2_v[pl.ds(off, 16)]
                    gt = jnp.logical_and(valid, kk > t_vec)
                    eq = jnp.logical_and(valid, kk == t_vec)
                    eqc = plsc.cumsum(eq.astype(jnp.int32))
                    take_eq = jnp.logical_and(eq, eqc <= budget_v)
                    take = jnp.logical_or(gt, take_eq)
                    pos = plsc.cumsum(take.astype(jnp.int32)) - 1 + cursor_v
                    idxs = cidx2_v[pl.ds(off, 16)]
                    plsc.store_scatter(out_v, [pos], idxs, mask=take)
                    cursor_v = (cursor_v
                                + plsc.all_reduce_population_count(take))
                    budget_v = (budget_v
                                - plsc.all_reduce_population_count(take_eq))
                return (cursor_v, budget_v)
            lax.fori_loop(0, (ncand + 1) // 2, fl,
                          (zeros, jnp.broadcast_to(need_l, (16,))))

            pltpu.sync_copy(out_v, out_hbm.at[b])

    return topk


def kernel(x, k_proj_w, queries):
    b, n, d = x.shape
    k = max(1, int(n * 0.1))
    nq, r = queries.shape

    blk = 1024
    rows_total = b * n
    nb = rows_total // blk
    x2 = x.reshape(rows_total, d)
    ts3 = pl.pallas_call(
        _score_body,
        grid=(nb,),
        in_specs=[
            pl.BlockSpec((blk, d), lambda i: (i, 0)),
            pl.BlockSpec((r, d), lambda i: (0, 0)),
            pl.BlockSpec((nq, r), lambda i: (0, 0)),
        ],
        out_specs=pl.BlockSpec((1, 1, blk), lambda i: (i, 0, 0)),
        out_shape=jax.ShapeDtypeStruct((nb, 1, blk), jnp.float32),
        scratch_shapes=[pltpu.VMEM((nq, d), jnp.float32)],
    )(x2, k_proj_w, queries)
    ts = ts3.reshape(b, n)

    out_pad = ((k + 7) // 8) * 8
    out = _make_topk(b, n, k, out_pad)(ts)
    return out[:, :k]
